# Initial kernel scaffold; baseline (speedup 1.0000x reference)
import jax
import jax.numpy as jnp
from jax.experimental import pallas as pl


def kernel(x, edge_data, node_trans, adj_mat, W_ih, W_hh, b_ih, b_hh, W_dnn, b_dnn, W_g1, att1, bias1, W_g2, att2, bias2, W_gcn1, b_gcn1, W_gcn2, b_gcn2, W_cls1, b_cls1, W_cls2, b_cls2, W_cls3, b_cls3, edge_index):
    raise NotImplementedError("write your pallas kernel here")


# speedup vs baseline: 12.4973x; 12.4973x over previous
import functools

import jax
import jax.numpy as jnp
from jax import lax
from jax.experimental import pallas as pl
from jax.experimental.pallas import tpu as pltpu
from jax.experimental.pallas import tpu_sc as plsc

N = 10000
E = 320000
LSEQ = 20
DF = 128
HID = 16

_NC = 2
_NS = 16
_NW = _NC * _NS
_CH = 128
_NCHUNK = E // _CH
_ITERS = -(-_NCHUNK // _NW)
_RPT = 624
_TAIL = N - _RPT * _NS

_RB = 1000
_GRID = N // _RB

def _deg_body(dstq, outp, idx_v, ones_v, zer_v, acc):
    cid = lax.axis_index("c")
    sid = lax.axis_index("s")
    wid = sid * _NC + cid

    def _init(r, carry):
        ones_v[r, :] = jnp.ones((16,), jnp.float32)
        return carry

    lax.fori_loop(0, _CH, _init, None)

    def _zero(r, carry):
        zer_v[r, :] = jnp.zeros((16,), jnp.float32)
        return carry

    lax.fori_loop(0, _RPT, _zero, None)
    pltpu.sync_copy(zer_v, acc.at[pl.ds(sid * _RPT, _RPT)])

    @pl.when(sid == 0)
    def _():
        pltpu.sync_copy(zer_v.at[pl.ds(0, _TAIL)],
                        acc.at[pl.ds(_RPT * _NS, _TAIL)])

    plsc.subcore_barrier()

    def _step(i, carry):
        chunk = i * _NW + wid

        @pl.when(chunk < _NCHUNK)
        def _():
            pltpu.sync_copy(dstq.at[chunk], idx_v)
            pltpu.sync_copy(ones_v, acc.at[idx_v], add=True)

        return carry

    lax.fori_loop(0, _ITERS, _step, None)
    plsc.subcore_barrier()
    pltpu.sync_copy(acc.at[pl.ds(sid * _RPT, _RPT)],
                    outp.at[cid, pl.ds(sid * _RPT, _RPT)])

    @pl.when(sid == 0)
    def _():
        pltpu.sync_copy(acc.at[pl.ds(_RPT * _NS, _TAIL)],
                        outp.at[cid, pl.ds(_RPT * _NS, _TAIL)])


@functools.lru_cache(maxsize=None)
def _deg_call():
    mesh = plsc.VectorSubcoreMesh(core_axis_name="c", subcore_axis_name="s",
                                  num_cores=_NC, num_subcores=_NS)
    return pl.kernel(
        _deg_body,
        out_type=jax.ShapeDtypeStruct((_NC, N, 16), jnp.float32),
        mesh=mesh,
        compiler_params=pltpu.CompilerParams(use_tc_tiling_on_sc=False),
        scratch_types=[
            pltpu.VMEM((_CH,), jnp.int32),
            pltpu.VMEM((_CH, 16), jnp.float32),
            pltpu.VMEM((_RPT, 16), jnp.float32),
            pltpu.VMEM_SHARED((N, 16), jnp.float32),
        ],
    )


def _agg_body(srcq, dstq, tab, outp, idx_s, idx_d, rows_v, zer_v, acc, sem):
    cid = lax.axis_index("c")
    sid = lax.axis_index("s")
    wid = sid * _NC + cid

    def _zero(r, carry):
        for cc in range(4):
            zer_v[r, pl.ds(cc * 16, 16)] = jnp.zeros((16,), jnp.float32)
        return carry

    lax.fori_loop(0, _RPT, _zero, None)
    pltpu.sync_copy(zer_v, acc.at[pl.ds(sid * _RPT, _RPT)])

    @pl.when(sid == 0)
    def _():
        pltpu.sync_copy(zer_v.at[pl.ds(0, _TAIL)],
                        acc.at[pl.ds(_RPT * _NS, _TAIL)])

    plsc.subcore_barrier()

    def _step(i, carry):
        chunk = i * _NW + wid

        @pl.when(chunk < _NCHUNK)
        def _():
            pltpu.sync_copy(srcq.at[chunk], idx_s)
            pltpu.sync_copy(dstq.at[chunk], idx_d)
            pltpu.async_copy(tab.at[idx_s], rows_v, sem).wait()
            pltpu.sync_copy(rows_v, acc.at[idx_d], add=True)

        return carry

    lax.fori_loop(0, _ITERS, _step, None)
    plsc.subcore_barrier()
    pltpu.sync_copy(acc.at[pl.ds(sid * _RPT, _RPT)],
                    outp.at[cid, pl.ds(sid * _RPT, _RPT)])

    @pl.when(sid == 0)
    def _():
        pltpu.sync_copy(acc.at[pl.ds(_RPT * _NS, _TAIL)],
                        outp.at[cid, pl.ds(_RPT * _NS, _TAIL)])


@functools.lru_cache(maxsize=None)
def _agg_call():
    mesh = plsc.VectorSubcoreMesh(core_axis_name="c", subcore_axis_name="s",
                                  num_cores=_NC, num_subcores=_NS)
    return pl.kernel(
        _agg_body,
        out_type=jax.ShapeDtypeStruct((_NC, N, 64), jnp.float32),
        mesh=mesh,
        compiler_params=pltpu.CompilerParams(use_tc_tiling_on_sc=False),
        scratch_types=[
            pltpu.VMEM((_CH,), jnp.int32),
            pltpu.VMEM((_CH,), jnp.int32),
            pltpu.VMEM((_CH, 64), jnp.float32),
            pltpu.VMEM((_RPT, 64), jnp.float32),
            pltpu.VMEM_SHARED((N, 64), jnp.float32),
            pltpu.SemaphoreType.DMA,
        ],
    )


def _leaky(v):
    return jnp.where(v >= 0.0, v, 0.01 * v)


def _stageB_kernel(x_ref, nt0_ref, nt1_ref, wihT_ref, whhT_ref, bg_ref,
                   degp_ref, wg1_ref, xf_ref, xs1_ref, self1_ref, dinv_ref):
    xb = x_ref[...]
    n0 = nt0_ref[...]
    n1 = nt1_ref[...]
    wihT = wihT_ref[...]
    whhT = whhT_ref[...]
    bg = bg_ref[...]
    w0 = wihT[0:1, :]
    w1 = wihT[1:2, :]
    h = jnp.zeros((_RB, HID), jnp.float32)
    c = jnp.zeros((_RB, HID), jnp.float32)
    for t in range(LSEQ):
        xt0 = lax.slice(n0, (0, t), (_RB, t + 1))
        xt1 = lax.slice(n1, (0, t), (_RB, t + 1))
        g = (xt0 * w0 + xt1 * w1 + bg
             + jnp.dot(h, whhT, preferred_element_type=jnp.float32))
        ig = jax.nn.sigmoid(g[:, 0:16])
        fg = jax.nn.sigmoid(g[:, 16:32])
        gg = jnp.tanh(g[:, 32:48])
        og = jax.nn.sigmoid(g[:, 48:64])
        c = fg * c + ig * gg
        h = og * jnp.tanh(c)
    xf = jnp.concatenate([xb, h], axis=1)
    xf_ref[...] = xf
    dp = degp_ref[...]
    deg = dp[0][:, 0:1] + dp[1][:, 0:1] + 1.0
    dinv = lax.rsqrt(deg)
    dinv_ref[...] = dinv
    xw = jnp.dot(xf, wg1_ref[...], preferred_element_type=jnp.float32)
    xs1_ref[...] = xw * dinv
    self1_ref[...] = xw * (dinv * dinv)


def _stageC_kernel(aggp_ref, self1_ref, dinv_ref, b1_ref, wg2_ref,
                   xs2_ref, self2_ref):
    ap = aggp_ref[...]
    dinv = dinv_ref[...]
    pre = dinv * (ap[0] + ap[1]) + self1_ref[...] + b1_ref[...]
    out1 = _leaky(pre)
    xw2 = jnp.dot(out1, wg2_ref[...], preferred_element_type=jnp.float32)
    xs2_ref[...] = xw2 * dinv
    self2_ref[...] = xw2 * (dinv * dinv)


def _stageD_kernel(aggp_ref, self2_ref, dinv_ref, b2_ref, xf_ref,
                   w1T_ref, b1_ref, w2T_ref, b2c_ref, w3T_ref, b3_ref,
                   out_ref, xgcn_ref):
    ap = aggp_ref[...]
    dinv = dinv_ref[...]
    pre = dinv * (ap[0] + ap[1]) + self2_ref[...] + b2_ref[...]
    xg = _leaky(pre)
    xgcn_ref[...] = xg
    xc = jnp.concatenate([xf_ref[...], xg], axis=1)
    h1 = _leaky(jnp.dot(xc, w1T_ref[...], preferred_element_type=jnp.float32)
                + b1_ref[...])
    h2 = _leaky(jnp.dot(h1, w2T_ref[...], preferred_element_type=jnp.float32)
                + b2c_ref[...])
    out_ref[...] = (jnp.dot(h2, w3T_ref[...], preferred_element_type=jnp.float32)
                    + b3_ref[...])


def _row_spec(cols):
    return pl.BlockSpec((_RB, cols), lambda i: (i, 0))


def _full_spec(shape):
    nd = len(shape)
    return pl.BlockSpec(shape, lambda i, _n=nd: (0,) * _n)


def _part_spec(cols):
    return pl.BlockSpec((_NC, _RB, cols), lambda i: (0, i, 0))


def _stageB(x, nt0, nt1, wihT, whhT, bg, degp, wg1):
    return pl.pallas_call(
        _stageB_kernel,
        grid=(_GRID,),
        in_specs=[
            _row_spec(DF), _row_spec(LSEQ), _row_spec(LSEQ),
            _full_spec((2, 64)), _full_spec((HID, 64)), _full_spec((1, 64)),
            _part_spec(16), _full_spec((DF + HID, 64)),
        ],
        out_specs=[_row_spec(DF + HID), _row_spec(64), _row_spec(64),
                   _row_spec(1)],
        out_shape=[
            jax.ShapeDtypeStruct((N, DF + HID), jnp.float32),
            jax.ShapeDtypeStruct((N, 64), jnp.float32),
            jax.ShapeDtypeStruct((N, 64), jnp.float32),
            jax.ShapeDtypeStruct((N, 1), jnp.float32),
        ],
    )(x, nt0, nt1, wihT, whhT, bg, degp, wg1)


def _stageC(aggp, self1, dinv, b1, wg2):
    return pl.pallas_call(
        _stageC_kernel,
        grid=(_GRID,),
        in_specs=[_part_spec(64), _row_spec(64), _row_spec(1),
                  _full_spec((1, 64)), _full_spec((64, 64))],
        out_specs=[_row_spec(64), _row_spec(64)],
        out_shape=[
            jax.ShapeDtypeStruct((N, 64), jnp.float32),
            jax.ShapeDtypeStruct((N, 64), jnp.float32),
        ],
    )(aggp, self1, dinv, b1, wg2)


def _stageD(aggp, self2, dinv, b2, xf, w1T, b1, w2T, b2c, w3T, b3):
    return pl.pallas_call(
        _stageD_kernel,
        grid=(_GRID,),
        in_specs=[_part_spec(64), _row_spec(64), _row_spec(1),
                  _full_spec((1, 64)), _row_spec(DF + HID),
                  _full_spec((DF + HID + 64, 32)), _full_spec((1, 32)),
                  _full_spec((32, 16)), _full_spec((1, 16)),
                  _full_spec((16, 4)), _full_spec((1, 4))],
        out_specs=[_row_spec(4), _row_spec(64)],
        out_shape=[
            jax.ShapeDtypeStruct((N, 4), jnp.float32),
            jax.ShapeDtypeStruct((N, 64), jnp.float32),
        ],
    )(aggp, self2, dinv, b2, xf, w1T, b1, w2T, b2c, w3T, b3)


def kernel(x, edge_data, node_trans, adj_mat, W_ih, W_hh, b_ih, b_hh,
           W_dnn, b_dnn, W_g1, att1, bias1, W_g2, att2, bias2,
           W_gcn1, b_gcn1, W_gcn2, b_gcn2, W_cls1, b_cls1, W_cls2, b_cls2,
           W_cls3, b_cls3, edge_index):
    srcq = edge_index[0].reshape(_NCHUNK, _CH)
    dstq = edge_index[1].reshape(_NCHUNK, _CH)
    nt0 = node_trans[:, :, 0]
    nt1 = node_trans[:, :, 1]
    wihT = W_ih.T
    whhT = W_hh.T
    bg = (b_ih + b_hh).reshape(1, 4 * HID)

    degp = _deg_call()(dstq)
    xf, xs1, self1, dinv = _stageB(x, nt0, nt1, wihT, whhT, bg, degp, W_gcn1)
    agg1 = _agg_call()(srcq, dstq, xs1)
    xs2, self2 = _stageC(agg1, self1, dinv, b_gcn1.reshape(1, 64), W_gcn2)
    agg2 = _agg_call()(srcq, dstq, xs2)
    out, xgcn = _stageD(agg2, self2, dinv, b_gcn2.reshape(1, 64), xf,
                        W_cls1.T, b_cls1.reshape(1, 32),
                        W_cls2.T, b_cls2.reshape(1, 16),
                        W_cls3.T, b_cls3.reshape(1, 4))
    return (out, xgcn, adj_mat)


# speedup vs baseline: 21.3384x; 1.7074x over previous
import functools

import jax
import jax.numpy as jnp
from jax import lax
from jax.experimental import pallas as pl
from jax.experimental.pallas import tpu as pltpu
from jax.experimental.pallas import tpu_sc as plsc

N = 10000
E = 320000
LSEQ = 20
DF = 128
HID = 16

_NC = 2
_NS = 16
_NW = _NC * _NS
_CH = 128
_NCHUNK = E // _CH
_ITERS = -(-_NCHUNK // _NW)
_XTRA = _NCHUNK - (_ITERS - 1) * _NW
_OUTER = -(-_ITERS // 3)
_QPAD = _NCHUNK + 4
_RPT = 624
_TAIL = N - _RPT * _NS

_RB = 1000
_GRID = N // _RB

def _tile_range(wid):
    start = 78 * wid + jnp.minimum(wid, _XTRA)
    cnt = jnp.where(wid < _XTRA, _ITERS, _ITERS - 1)
    return start, cnt


def _deg_body(dstq, outp, idxd_v, ones_v, zer_v, acc, ssem):
    cid = lax.axis_index("c")
    sid = lax.axis_index("s")
    wid = sid * _NC + cid
    start, cnt = _tile_range(wid)

    def _init(r, carry):
        ones_v[r, :] = jnp.ones((16,), jnp.float32)
        return carry

    lax.fori_loop(0, _CH, _init, None)

    def _zero(r, carry):
        zer_v[r, :] = jnp.zeros((16,), jnp.float32)
        return carry

    lax.fori_loop(0, _RPT, _zero, None)
    pltpu.sync_copy(dstq.at[pl.ds(start, _ITERS)], idxd_v)
    pltpu.sync_copy(zer_v, acc.at[pl.ds(sid * _RPT, _RPT)])

    @pl.when(sid == 0)
    def _():
        pltpu.sync_copy(zer_v.at[pl.ds(0, _TAIL)],
                        acc.at[pl.ds(_RPT * _NS, _TAIL)])

    plsc.subcore_barrier()

    def _outer(ii, carry):
        for b in range(3):
            i = ii * 3 + b

            @pl.when(i < cnt)
            def _():
                pltpu.async_copy(ones_v, acc.at[idxd_v.at[i]], ssem,
                                 add=True)

        for b in range(3):
            i = ii * 3 + b

            @pl.when(i < cnt)
            def _():
                pltpu.make_async_copy(ones_v, acc.at[idxd_v.at[i]],
                                      ssem).wait()

        return carry

    lax.fori_loop(0, _OUTER, _outer, None)
    plsc.subcore_barrier()
    pltpu.sync_copy(acc.at[pl.ds(sid * _RPT, _RPT)],
                    outp.at[cid, pl.ds(sid * _RPT, _RPT)])

    @pl.when(sid == 0)
    def _():
        pltpu.sync_copy(acc.at[pl.ds(_RPT * _NS, _TAIL)],
                        outp.at[cid, pl.ds(_RPT * _NS, _TAIL)])


@functools.lru_cache(maxsize=None)
def _deg_call():
    mesh = plsc.VectorSubcoreMesh(core_axis_name="c", subcore_axis_name="s",
                                  num_cores=_NC, num_subcores=_NS)
    return pl.kernel(
        _deg_body,
        out_type=jax.ShapeDtypeStruct((_NC, N, 16), jnp.float32),
        mesh=mesh,
        compiler_params=pltpu.CompilerParams(use_tc_tiling_on_sc=False),
        scratch_types=[
            pltpu.VMEM((_ITERS, _CH), jnp.int32),
            pltpu.VMEM((_CH, 16), jnp.float32),
            pltpu.VMEM((_RPT, 16), jnp.float32),
            pltpu.VMEM_SHARED((N, 16), jnp.float32),
            pltpu.SemaphoreType.DMA,
        ],
    )


def _agg_body(srcq, dstq, tab, outp, idxs_v, idxd_v,
              rows0, rows1, rows2, zer_v, acc, sem0, sem1, sem2):
    cid = lax.axis_index("c")
    sid = lax.axis_index("s")
    wid = sid * _NC + cid
    start, cnt = _tile_range(wid)
    rows = (rows0, rows1, rows2)
    sems = (sem0, sem1, sem2)

    def _zero(r, carry):
        for cc in range(4):
            zer_v[r, pl.ds(cc * 16, 16)] = jnp.zeros((16,), jnp.float32)
        return carry

    lax.fori_loop(0, _RPT, _zero, None)
    pltpu.sync_copy(srcq.at[pl.ds(start, _ITERS)], idxs_v)
    pltpu.sync_copy(dstq.at[pl.ds(start, _ITERS)], idxd_v)
    pltpu.sync_copy(zer_v, acc.at[pl.ds(sid * _RPT, _RPT)])

    @pl.when(sid == 0)
    def _():
        pltpu.sync_copy(zer_v.at[pl.ds(0, _TAIL)],
                        acc.at[pl.ds(_RPT * _NS, _TAIL)])

    plsc.subcore_barrier()

    for b in range(3):
        pltpu.async_copy(tab.at[idxs_v.at[b]], rows[b], sems[b])

    def _outer(ii, carry):
        for b in range(3):
            i = ii * 3 + b

            @pl.when(i < cnt)
            def _():
                pltpu.make_async_copy(tab.at[idxs_v.at[i]], rows[b],
                                      sems[b]).wait()
                pltpu.sync_copy(rows[b], acc.at[idxd_v.at[i]], add=True)

                @pl.when(i + 3 < cnt)
                def _():
                    pltpu.async_copy(tab.at[idxs_v.at[i + 3]], rows[b],
                                     sems[b])

        return carry

    lax.fori_loop(0, _OUTER, _outer, None)
    plsc.subcore_barrier()
    pltpu.sync_copy(acc.at[pl.ds(sid * _RPT, _RPT)],
                    outp.at[cid, pl.ds(sid * _RPT, _RPT)])

    @pl.when(sid == 0)
    def _():
        pltpu.sync_copy(acc.at[pl.ds(_RPT * _NS, _TAIL)],
                        outp.at[cid, pl.ds(_RPT * _NS, _TAIL)])


@functools.lru_cache(maxsize=None)
def _agg_call():
    mesh = plsc.VectorSubcoreMesh(core_axis_name="c", subcore_axis_name="s",
                                  num_cores=_NC, num_subcores=_NS)
    return pl.kernel(
        _agg_body,
        out_type=jax.ShapeDtypeStruct((_NC, N, 64), jnp.float32),
        mesh=mesh,
        compiler_params=pltpu.CompilerParams(use_tc_tiling_on_sc=False),
        scratch_types=[
            pltpu.VMEM((_ITERS, _CH), jnp.int32),
            pltpu.VMEM((_ITERS, _CH), jnp.int32),
            pltpu.VMEM((_CH, 64), jnp.float32),
            pltpu.VMEM((_CH, 64), jnp.float32),
            pltpu.VMEM((_CH, 64), jnp.float32),
            pltpu.VMEM((_RPT, 64), jnp.float32),
            pltpu.VMEM_SHARED((N, 64), jnp.float32),
            pltpu.SemaphoreType.DMA,
            pltpu.SemaphoreType.DMA,
            pltpu.SemaphoreType.DMA,
        ],
    )


def _leaky(v):
    return jnp.where(v >= 0.0, v, 0.01 * v)


def _stageB_kernel(x_ref, nt0_ref, nt1_ref, wihT_ref, whhT_ref, bg_ref,
                   degp_ref, wg1_ref, xf_ref, xs1_ref, self1_ref, dinv_ref):
    xb = x_ref[...]
    n0 = nt0_ref[...]
    n1 = nt1_ref[...]
    wihT = wihT_ref[...]
    whhT = whhT_ref[...]
    bg = bg_ref[...]
    w0 = wihT[0:1, :]
    w1 = wihT[1:2, :]
    h = jnp.zeros((_RB, HID), jnp.float32)
    c = jnp.zeros((_RB, HID), jnp.float32)
    for t in range(LSEQ):
        xt0 = lax.slice(n0, (0, t), (_RB, t + 1))
        xt1 = lax.slice(n1, (0, t), (_RB, t + 1))
        g = (xt0 * w0 + xt1 * w1 + bg
             + jnp.dot(h, whhT, preferred_element_type=jnp.float32))
        ig = jax.nn.sigmoid(g[:, 0:16])
        fg = jax.nn.sigmoid(g[:, 16:32])
        gg = jnp.tanh(g[:, 32:48])
        og = jax.nn.sigmoid(g[:, 48:64])
        c = fg * c + ig * gg
        h = og * jnp.tanh(c)
    xf = jnp.concatenate([xb, h], axis=1)
    xf_ref[...] = xf
    dp = degp_ref[...]
    deg = dp[0][:, 0:1] + dp[1][:, 0:1] + 1.0
    dinv = lax.rsqrt(deg)
    dinv_ref[...] = dinv
    xw = jnp.dot(xf, wg1_ref[...], preferred_element_type=jnp.float32)
    xs1_ref[...] = xw * dinv
    self1_ref[...] = xw * (dinv * dinv)


def _stageC_kernel(aggp_ref, self1_ref, dinv_ref, b1_ref, wg2_ref,
                   xs2_ref, self2_ref):
    ap = aggp_ref[...]
    dinv = dinv_ref[...]
    pre = dinv * (ap[0] + ap[1]) + self1_ref[...] + b1_ref[...]
    out1 = _leaky(pre)
    xw2 = jnp.dot(out1, wg2_ref[...], preferred_element_type=jnp.float32)
    xs2_ref[...] = xw2 * dinv
    self2_ref[...] = xw2 * (dinv * dinv)


def _stageD_kernel(aggp_ref, self2_ref, dinv_ref, b2_ref, xf_ref,
                   w1T_ref, b1_ref, w2T_ref, b2c_ref, w3T_ref, b3_ref,
                   out_ref, xgcn_ref):
    ap = aggp_ref[...]
    dinv = dinv_ref[...]
    pre = dinv * (ap[0] + ap[1]) + self2_ref[...] + b2_ref[...]
    xg = _leaky(pre)
    xgcn_ref[...] = xg
    xc = jnp.concatenate([xf_ref[...], xg], axis=1)
    h1 = _leaky(jnp.dot(xc, w1T_ref[...], preferred_element_type=jnp.float32)
                + b1_ref[...])
    h2 = _leaky(jnp.dot(h1, w2T_ref[...], preferred_element_type=jnp.float32)
                + b2c_ref[...])
    out_ref[...] = (jnp.dot(h2, w3T_ref[...], preferred_element_type=jnp.float32)
                    + b3_ref[...])


def _row_spec(cols):
    return pl.BlockSpec((_RB, cols), lambda i: (i, 0))


def _full_spec(shape):
    nd = len(shape)
    return pl.BlockSpec(shape, lambda i, _n=nd: (0,) * _n)


def _part_spec(cols):
    return pl.BlockSpec((_NC, _RB, cols), lambda i: (0, i, 0))


def _stageB(x, nt0, nt1, wihT, whhT, bg, degp, wg1):
    return pl.pallas_call(
        _stageB_kernel,
        grid=(_GRID,),
        in_specs=[
            _row_spec(DF), _row_spec(LSEQ), _row_spec(LSEQ),
            _full_spec((2, 64)), _full_spec((HID, 64)), _full_spec((1, 64)),
            _part_spec(16), _full_spec((DF + HID, 64)),
        ],
        out_specs=[_row_spec(DF + HID), _row_spec(64), _row_spec(64),
                   _row_spec(1)],
        out_shape=[
            jax.ShapeDtypeStruct((N, DF + HID), jnp.float32),
            jax.ShapeDtypeStruct((N, 64), jnp.float32),
            jax.ShapeDtypeStruct((N, 64), jnp.float32),
            jax.ShapeDtypeStruct((N, 1), jnp.float32),
        ],
    )(x, nt0, nt1, wihT, whhT, bg, degp, wg1)


def _stageC(aggp, self1, dinv, b1, wg2):
    return pl.pallas_call(
        _stageC_kernel,
        grid=(_GRID,),
        in_specs=[_part_spec(64), _row_spec(64), _row_spec(1),
                  _full_spec((1, 64)), _full_spec((64, 64))],
        out_specs=[_row_spec(64), _row_spec(64)],
        out_shape=[
            jax.ShapeDtypeStruct((N, 64), jnp.float32),
            jax.ShapeDtypeStruct((N, 64), jnp.float32),
        ],
    )(aggp, self1, dinv, b1, wg2)


def _stageD(aggp, self2, dinv, b2, xf, w1T, b1, w2T, b2c, w3T, b3):
    return pl.pallas_call(
        _stageD_kernel,
        grid=(_GRID,),
        in_specs=[_part_spec(64), _row_spec(64), _row_spec(1),
                  _full_spec((1, 64)), _row_spec(DF + HID),
                  _full_spec((DF + HID + 64, 32)), _full_spec((1, 32)),
                  _full_spec((32, 16)), _full_spec((1, 16)),
                  _full_spec((16, 4)), _full_spec((1, 4))],
        out_specs=[_row_spec(4), _row_spec(64)],
        out_shape=[
            jax.ShapeDtypeStruct((N, 4), jnp.float32),
            jax.ShapeDtypeStruct((N, 64), jnp.float32),
        ],
    )(aggp, self2, dinv, b2, xf, w1T, b1, w2T, b2c, w3T, b3)


def kernel(x, edge_data, node_trans, adj_mat, W_ih, W_hh, b_ih, b_hh,
           W_dnn, b_dnn, W_g1, att1, bias1, W_g2, att2, bias2,
           W_gcn1, b_gcn1, W_gcn2, b_gcn2, W_cls1, b_cls1, W_cls2, b_cls2,
           W_cls3, b_cls3, edge_index):
    srcq = jnp.pad(edge_index[0].reshape(_NCHUNK, _CH),
                   ((0, _QPAD - _NCHUNK), (0, 0)))
    dstq = jnp.pad(edge_index[1].reshape(_NCHUNK, _CH),
                   ((0, _QPAD - _NCHUNK), (0, 0)))
    nt0 = node_trans[:, :, 0]
    nt1 = node_trans[:, :, 1]
    wihT = W_ih.T
    whhT = W_hh.T
    bg = (b_ih + b_hh).reshape(1, 4 * HID)

    degp = _deg_call()(dstq)
    xf, xs1, self1, dinv = _stageB(x, nt0, nt1, wihT, whhT, bg, degp, W_gcn1)
    agg1 = _agg_call()(srcq, dstq, xs1)
    xs2, self2 = _stageC(agg1, self1, dinv, b_gcn1.reshape(1, 64), W_gcn2)
    agg2 = _agg_call()(srcq, dstq, xs2)
    out, xgcn = _stageD(agg2, self2, dinv, b_gcn2.reshape(1, 64), xf,
                        W_cls1.T, b_cls1.reshape(1, 32),
                        W_cls2.T, b_cls2.reshape(1, 16),
                        W_cls3.T, b_cls3.reshape(1, 4))
    return (out, xgcn, adj_mat)


# speedup vs baseline: 33.9990x; 1.5933x over previous
import functools

import jax
import jax.numpy as jnp
from jax import lax
from jax.experimental import pallas as pl
from jax.experimental.pallas import tpu as pltpu
from jax.experimental.pallas import tpu_sc as plsc

N = 10000
E = 320000
LSEQ = 20
DF = 128
HID = 16

_NC = 2
_NS = 16
_NW = _NC * _NS
_CH = 128
_NCHUNK = E // _CH
_ITERS = -(-_NCHUNK // _NW)
_XTRA = _NCHUNK - (_ITERS - 1) * _NW
_OUTER = -(-_ITERS // 3)
_QPAD = _NCHUNK + 4
_RPT = 624
_TAIL = N - _RPT * _NS

_RB = 1000
_GRID = N // _RB

def _tile_range(wid):
    start = 78 * wid + jnp.minimum(wid, _XTRA)
    cnt = jnp.where(wid < _XTRA, _ITERS, _ITERS - 1)
    return start, cnt


def _deg_body(dstq, outp, idxd_v, ones_v, zer_v, acc, ssem):
    cid = lax.axis_index("c")
    sid = lax.axis_index("s")
    wid = sid * _NC + cid
    start, cnt = _tile_range(wid)

    def _init(r, carry):
        ones_v[r, :] = jnp.ones((16,), jnp.float32)
        return carry

    lax.fori_loop(0, _CH, _init, None)

    def _zero(r, carry):
        zer_v[r, :] = jnp.zeros((16,), jnp.float32)
        return carry

    lax.fori_loop(0, _RPT, _zero, None)
    pltpu.sync_copy(dstq.at[pl.ds(start, _ITERS)], idxd_v)
    pltpu.sync_copy(zer_v, acc.at[pl.ds(sid * _RPT, _RPT)])

    @pl.when(sid == 0)
    def _():
        pltpu.sync_copy(zer_v.at[pl.ds(0, _TAIL)],
                        acc.at[pl.ds(_RPT * _NS, _TAIL)])

    plsc.subcore_barrier()

    def _outer(ii, carry):
        for b in range(3):
            i = ii * 3 + b

            @pl.when(i < cnt)
            def _():
                pltpu.async_copy(ones_v, acc.at[idxd_v.at[i]], ssem,
                                 add=True)

        for b in range(3):
            i = ii * 3 + b

            @pl.when(i < cnt)
            def _():
                pltpu.make_async_copy(ones_v, acc.at[idxd_v.at[i]],
                                      ssem).wait()

        return carry

    lax.fori_loop(0, _OUTER, _outer, None)
    plsc.subcore_barrier()
    pltpu.sync_copy(acc.at[pl.ds(sid * _RPT, _RPT)],
                    outp.at[cid, pl.ds(sid * _RPT, _RPT)])

    @pl.when(sid == 0)
    def _():
        pltpu.sync_copy(acc.at[pl.ds(_RPT * _NS, _TAIL)],
                        outp.at[cid, pl.ds(_RPT * _NS, _TAIL)])


@functools.lru_cache(maxsize=None)
def _deg_call():
    mesh = plsc.VectorSubcoreMesh(core_axis_name="c", subcore_axis_name="s",
                                  num_cores=_NC, num_subcores=_NS)
    return pl.kernel(
        _deg_body,
        out_type=jax.ShapeDtypeStruct((_NC, N, 16), jnp.float32),
        mesh=mesh,
        compiler_params=pltpu.CompilerParams(use_tc_tiling_on_sc=False),
        scratch_types=[
            pltpu.VMEM((_ITERS, _CH), jnp.int32),
            pltpu.VMEM((_CH, 16), jnp.float32),
            pltpu.VMEM((_RPT, 16), jnp.float32),
            pltpu.VMEM_SHARED((N, 16), jnp.float32),
            pltpu.SemaphoreType.DMA,
        ],
    )


def _agg_body(srcq, dstq, tab, outp, idxs_v, idxd_v,
              rows0, rows1, rows2, zer_v, acc, sem0, sem1, sem2):
    cid = lax.axis_index("c")
    sid = lax.axis_index("s")
    wid = sid * _NC + cid
    start, cnt = _tile_range(wid)
    rows = (rows0, rows1, rows2)
    sems = (sem0, sem1, sem2)

    def _zero(r, carry):
        for cc in range(4):
            zer_v[r, pl.ds(cc * 16, 16)] = jnp.zeros((16,), jnp.float32)
        return carry

    lax.fori_loop(0, _RPT, _zero, None)
    pltpu.sync_copy(srcq.at[pl.ds(start, _ITERS)], idxs_v)
    pltpu.sync_copy(dstq.at[pl.ds(start, _ITERS)], idxd_v)
    pltpu.sync_copy(zer_v, acc.at[pl.ds(sid * _RPT, _RPT)])

    @pl.when(sid == 0)
    def _():
        pltpu.sync_copy(zer_v.at[pl.ds(0, _TAIL)],
                        acc.at[pl.ds(_RPT * _NS, _TAIL)])

    plsc.subcore_barrier()

    for b in range(3):
        pltpu.async_copy(tab.at[idxs_v.at[b]], rows[b], sems[b])

    def _outer(ii, carry):
        for b in range(3):
            i = ii * 3 + b

            @pl.when(i < cnt)
            def _():
                pltpu.make_async_copy(tab.at[idxs_v.at[i]], rows[b],
                                      sems[b]).wait()
                pltpu.sync_copy(rows[b], acc.at[idxd_v.at[i]], add=True)

                @pl.when(i + 3 < cnt)
                def _():
                    pltpu.async_copy(tab.at[idxs_v.at[i + 3]], rows[b],
                                     sems[b])

        return carry

    lax.fori_loop(0, _OUTER, _outer, None)
    plsc.subcore_barrier()
    pltpu.sync_copy(acc.at[pl.ds(sid * _RPT, _RPT)],
                    outp.at[cid, pl.ds(sid * _RPT, _RPT)])

    @pl.when(sid == 0)
    def _():
        pltpu.sync_copy(acc.at[pl.ds(_RPT * _NS, _TAIL)],
                        outp.at[cid, pl.ds(_RPT * _NS, _TAIL)])


@functools.lru_cache(maxsize=None)
def _agg_call():
    mesh = plsc.VectorSubcoreMesh(core_axis_name="c", subcore_axis_name="s",
                                  num_cores=_NC, num_subcores=_NS)
    return pl.kernel(
        _agg_body,
        out_type=jax.ShapeDtypeStruct((_NC, N, 64), jnp.float32),
        mesh=mesh,
        compiler_params=pltpu.CompilerParams(use_tc_tiling_on_sc=False),
        scratch_types=[
            pltpu.VMEM((_ITERS, _CH), jnp.int32),
            pltpu.VMEM((_ITERS, _CH), jnp.int32),
            pltpu.VMEM((_CH, 64), jnp.float32),
            pltpu.VMEM((_CH, 64), jnp.float32),
            pltpu.VMEM((_CH, 64), jnp.float32),
            pltpu.VMEM((_RPT, 64), jnp.float32),
            pltpu.VMEM_SHARED((N, 64), jnp.float32),
            pltpu.SemaphoreType.DMA,
            pltpu.SemaphoreType.DMA,
            pltpu.SemaphoreType.DMA,
        ],
    )


def _leaky(v):
    return jnp.where(v >= 0.0, v, 0.01 * v)


_CB = 2048
_NPAD = 10240


def _lstm_kernel(nt0_ref, nt1_ref, w0_ref, w1_ref, whh_ref, bg_ref, hT_ref):
    n0 = nt0_ref[...]
    n1 = nt1_ref[...]
    w0 = w0_ref[...]
    w1 = w1_ref[...]
    whh = whh_ref[...]
    bg = bg_ref[...]
    h = jnp.zeros((HID, _CB), jnp.float32)
    c = jnp.zeros((HID, _CB), jnp.float32)
    for t in range(LSEQ):
        xt0 = lax.slice(n0, (t, 0), (t + 1, _CB))
        xt1 = lax.slice(n1, (t, 0), (t + 1, _CB))
        g = (w0 * xt0 + w1 * xt1 + bg
             + jnp.dot(whh, h, preferred_element_type=jnp.float32))
        ig = jax.nn.sigmoid(g[0:16])
        fg = jax.nn.sigmoid(g[16:32])
        gg = jnp.tanh(g[32:48])
        og = jax.nn.sigmoid(g[48:64])
        c = fg * c + ig * gg
        h = og * jnp.tanh(c)
    hT_ref[...] = h


def _stageB2_kernel(x_ref, rn_ref, degp_ref, w1x_ref, w1h_ref,
                    xs1_ref, self1_ref, dinv_ref):
    dp = degp_ref[...]
    deg = dp[0][:, 0:1] + dp[1][:, 0:1] + 1.0
    dinv = lax.rsqrt(deg)
    dinv_ref[...] = dinv
    xw = (jnp.dot(x_ref[...], w1x_ref[...], preferred_element_type=jnp.float32)
          + jnp.dot(rn_ref[...], w1h_ref[...],
                    preferred_element_type=jnp.float32))
    xs1_ref[...] = xw * dinv
    self1_ref[...] = xw * (dinv * dinv)


def _stageC_kernel(aggp_ref, self1_ref, dinv_ref, b1_ref, wg2_ref,
                   xs2_ref, self2_ref):
    ap = aggp_ref[...]
    dinv = dinv_ref[...]
    pre = dinv * (ap[0] + ap[1]) + self1_ref[...] + b1_ref[...]
    out1 = _leaky(pre)
    xw2 = jnp.dot(out1, wg2_ref[...], preferred_element_type=jnp.float32)
    xs2_ref[...] = xw2 * dinv
    self2_ref[...] = xw2 * (dinv * dinv)


def _stageD_kernel(aggp_ref, self2_ref, dinv_ref, b2_ref, x_ref, rn_ref,
                   wcx_ref, wch_ref, wcg_ref, b1_ref, w2T_ref, b2c_ref,
                   w3T_ref, b3_ref, out_ref, xgcn_ref):
    ap = aggp_ref[...]
    dinv = dinv_ref[...]
    pre = dinv * (ap[0] + ap[1]) + self2_ref[...] + b2_ref[...]
    xg = _leaky(pre)
    xgcn_ref[...] = xg
    h1 = _leaky(jnp.dot(x_ref[...], wcx_ref[...],
                        preferred_element_type=jnp.float32)
                + jnp.dot(rn_ref[...], wch_ref[...],
                          preferred_element_type=jnp.float32)
                + jnp.dot(xg, wcg_ref[...],
                          preferred_element_type=jnp.float32)
                + b1_ref[...])
    h2 = _leaky(jnp.dot(h1, w2T_ref[...], preferred_element_type=jnp.float32)
                + b2c_ref[...])
    out_ref[...] = (jnp.dot(h2, w3T_ref[...], preferred_element_type=jnp.float32)
                    + b3_ref[...])


def _row_spec(cols):
    return pl.BlockSpec((_RB, cols), lambda i: (i, 0))


def _full_spec(shape):
    nd = len(shape)
    return pl.BlockSpec(shape, lambda i, _n=nd: (0,) * _n)


def _part_spec(cols):
    return pl.BlockSpec((_NC, _RB, cols), lambda i: (0, i, 0))


def _lstm(nt0T, nt1T, w0, w1, whh, bgc):
    return pl.pallas_call(
        _lstm_kernel,
        grid=(_NPAD // _CB,),
        in_specs=[
            pl.BlockSpec((LSEQ, _CB), lambda i: (0, i)),
            pl.BlockSpec((LSEQ, _CB), lambda i: (0, i)),
            _full_spec((4 * HID, 1)), _full_spec((4 * HID, 1)),
            _full_spec((4 * HID, HID)), _full_spec((4 * HID, 1)),
        ],
        out_specs=pl.BlockSpec((HID, _CB), lambda i: (0, i)),
        out_shape=jax.ShapeDtypeStruct((HID, _NPAD), jnp.float32),
    )(nt0T, nt1T, w0, w1, whh, bgc)


def _stageB2(x, rn, degp, w1x, w1h):
    return pl.pallas_call(
        _stageB2_kernel,
        grid=(_GRID,),
        in_specs=[
            _row_spec(DF), _row_spec(HID), _part_spec(16),
            _full_spec((DF, 64)), _full_spec((HID, 64)),
        ],
        out_specs=[_row_spec(64), _row_spec(64), _row_spec(1)],
        out_shape=[
            jax.ShapeDtypeStruct((N, 64), jnp.float32),
            jax.ShapeDtypeStruct((N, 64), jnp.float32),
            jax.ShapeDtypeStruct((N, 1), jnp.float32),
        ],
    )(x, rn, degp, w1x, w1h)


def _stageC(aggp, self1, dinv, b1, wg2):
    return pl.pallas_call(
        _stageC_kernel,
        grid=(_GRID,),
        in_specs=[_part_spec(64), _row_spec(64), _row_spec(1),
                  _full_spec((1, 64)), _full_spec((64, 64))],
        out_specs=[_row_spec(64), _row_spec(64)],
        out_shape=[
            jax.ShapeDtypeStruct((N, 64), jnp.float32),
            jax.ShapeDtypeStruct((N, 64), jnp.float32),
        ],
    )(aggp, self1, dinv, b1, wg2)


def _stageD(aggp, self2, dinv, b2, x, rn, wcx, wch, wcg, b1, w2T, b2c,
            w3T, b3):
    return pl.pallas_call(
        _stageD_kernel,
        grid=(_GRID,),
        in_specs=[_part_spec(64), _row_spec(64), _row_spec(1),
                  _full_spec((1, 64)), _row_spec(DF), _row_spec(HID),
                  _full_spec((DF, 32)), _full_spec((HID, 32)),
                  _full_spec((64, 32)), _full_spec((1, 32)),
                  _full_spec((32, 16)), _full_spec((1, 16)),
                  _full_spec((16, 4)), _full_spec((1, 4))],
        out_specs=[_row_spec(4), _row_spec(64)],
        out_shape=[
            jax.ShapeDtypeStruct((N, 4), jnp.float32),
            jax.ShapeDtypeStruct((N, 64), jnp.float32),
        ],
    )(aggp, self2, dinv, b2, x, rn, wcx, wch, wcg, b1, w2T, b2c, w3T, b3)


def kernel(x, edge_data, node_trans, adj_mat, W_ih, W_hh, b_ih, b_hh,
           W_dnn, b_dnn, W_g1, att1, bias1, W_g2, att2, bias2,
           W_gcn1, b_gcn1, W_gcn2, b_gcn2, W_cls1, b_cls1, W_cls2, b_cls2,
           W_cls3, b_cls3, edge_index):
    srcq = jnp.pad(edge_index[0].reshape(_NCHUNK, _CH),
                   ((0, _QPAD - _NCHUNK), (0, 0)))
    dstq = jnp.pad(edge_index[1].reshape(_NCHUNK, _CH),
                   ((0, _QPAD - _NCHUNK), (0, 0)))
    nt0T = jnp.pad(node_trans[:, :, 0].T, ((0, 0), (0, _NPAD - N)))
    nt1T = jnp.pad(node_trans[:, :, 1].T, ((0, 0), (0, _NPAD - N)))
    w0 = W_ih[:, 0:1]
    w1 = W_ih[:, 1:2]
    bgc = (b_ih + b_hh).reshape(4 * HID, 1)
    wcls1T = W_cls1.T

    degp = _deg_call()(dstq)
    rn = _lstm(nt0T, nt1T, w0, w1, W_hh, bgc)[:, :N].T
    xs1, self1, dinv = _stageB2(x, rn, degp, W_gcn1[:DF], W_gcn1[DF:])
    agg1 = _agg_call()(srcq, dstq, xs1)
    xs2, self2 = _stageC(agg1, self1, dinv, b_gcn1.reshape(1, 64), W_gcn2)
    agg2 = _agg_call()(srcq, dstq, xs2)
    out, xgcn = _stageD(agg2, self2, dinv, b_gcn2.reshape(1, 64), x, rn,
                        wcls1T[:DF], wcls1T[DF:DF + HID],
                        wcls1T[DF + HID:], b_cls1.reshape(1, 32),
                        W_cls2.T, b_cls2.reshape(1, 16),
                        W_cls3.T, b_cls3.reshape(1, 4))
    return (out, xgcn, adj_mat)


# speedup vs baseline: 34.2583x; 1.0076x over previous
import functools

import jax
import jax.numpy as jnp
from jax import lax
from jax.experimental import pallas as pl
from jax.experimental.pallas import tpu as pltpu
from jax.experimental.pallas import tpu_sc as plsc

N = 10000
E = 320000
LSEQ = 20
DF = 128
HID = 16

_NC = 2
_NS = 16
_NW = _NC * _NS
_CH = 128
_NCHUNK = E // _CH
_ITERS = -(-_NCHUNK // _NW)
_XTRA = _NCHUNK - (_ITERS - 1) * _NW
_OUTER = -(-_ITERS // 3)
_QPAD = _NCHUNK + 4
_RPT = 624
_TAIL = N - _RPT * _NS

_RB = 1000
_GRID = N // _RB

def _tile_range(wid):
    start = 78 * wid + jnp.minimum(wid, _XTRA)
    cnt = jnp.where(wid < _XTRA, _ITERS, _ITERS - 1)
    return start, cnt


def _deg_body(dstq, outp, idxd0, idxd1, idxd2, ones_v, zer_v, acc,
              isem0, isem1, isem2):
    cid = lax.axis_index("c")
    sid = lax.axis_index("s")
    wid = sid * _NC + cid
    start, cnt = _tile_range(wid)
    idxd = (idxd0, idxd1, idxd2)
    isems = (isem0, isem1, isem2)

    def _init(r, carry):
        ones_v[r, :] = jnp.ones((16,), jnp.float32)
        return carry

    lax.fori_loop(0, _CH, _init, None)

    def _zero(r, carry):
        zer_v[r, :] = jnp.zeros((16,), jnp.float32)
        return carry

    lax.fori_loop(0, _RPT, _zero, None)
    pltpu.sync_copy(zer_v, acc.at[pl.ds(sid * _RPT, _RPT)])

    @pl.when(sid == 0)
    def _():
        pltpu.sync_copy(zer_v.at[pl.ds(0, _TAIL)],
                        acc.at[pl.ds(_RPT * _NS, _TAIL)])

    plsc.subcore_barrier()

    for b in range(3):
        pltpu.async_copy(dstq.at[start + b], idxd[b], isems[b])

    def _outer(ii, carry):
        for b in range(3):
            i = ii * 3 + b

            @pl.when(i < cnt)
            def _():
                pltpu.make_async_copy(dstq.at[start + i], idxd[b],
                                      isems[b]).wait()
                pltpu.sync_copy(ones_v, acc.at[idxd[b]], add=True)

                @pl.when(i + 3 < cnt)
                def _():
                    pltpu.async_copy(dstq.at[start + i + 3], idxd[b],
                                     isems[b])

        return carry

    lax.fori_loop(0, _OUTER, _outer, None)
    plsc.subcore_barrier()
    pltpu.sync_copy(acc.at[pl.ds(sid * _RPT, _RPT)],
                    outp.at[cid, pl.ds(sid * _RPT, _RPT)])

    @pl.when(sid == 0)
    def _():
        pltpu.sync_copy(acc.at[pl.ds(_RPT * _NS, _TAIL)],
                        outp.at[cid, pl.ds(_RPT * _NS, _TAIL)])


@functools.lru_cache(maxsize=None)
def _deg_call():
    mesh = plsc.VectorSubcoreMesh(core_axis_name="c", subcore_axis_name="s",
                                  num_cores=_NC, num_subcores=_NS)
    return pl.kernel(
        _deg_body,
        out_type=jax.ShapeDtypeStruct((_NC, N, 16), jnp.float32),
        mesh=mesh,
        compiler_params=pltpu.CompilerParams(use_tc_tiling_on_sc=False),
        scratch_types=[
            pltpu.VMEM((_CH,), jnp.int32),
            pltpu.VMEM((_CH,), jnp.int32),
            pltpu.VMEM((_CH,), jnp.int32),
            pltpu.VMEM((_CH, 16), jnp.float32),
            pltpu.VMEM((_RPT, 16), jnp.float32),
            pltpu.VMEM_SHARED((N, 16), jnp.float32),
            pltpu.SemaphoreType.DMA,
            pltpu.SemaphoreType.DMA,
            pltpu.SemaphoreType.DMA,
        ],
    )


def _agg_body(srcq, dstq, tab, outp, idxs_v, idxd0, idxd1, idxd2,
              rows0, rows1, rows2, zer_v, acc,
              sem0, sem1, sem2, isem0, isem1, isem2):
    cid = lax.axis_index("c")
    sid = lax.axis_index("s")
    wid = sid * _NC + cid
    start, cnt = _tile_range(wid)
    rows = (rows0, rows1, rows2)
    sems = (sem0, sem1, sem2)
    idxd = (idxd0, idxd1, idxd2)
    isems = (isem0, isem1, isem2)

    def _zero(r, carry):
        for cc in range(4):
            zer_v[r, pl.ds(cc * 16, 16)] = jnp.zeros((16,), jnp.float32)
        return carry

    lax.fori_loop(0, _RPT, _zero, None)
    pltpu.sync_copy(srcq.at[pl.ds(start, _ITERS)], idxs_v)
    pltpu.sync_copy(zer_v, acc.at[pl.ds(sid * _RPT, _RPT)])

    @pl.when(sid == 0)
    def _():
        pltpu.sync_copy(zer_v.at[pl.ds(0, _TAIL)],
                        acc.at[pl.ds(_RPT * _NS, _TAIL)])

    plsc.subcore_barrier()

    for b in range(3):
        pltpu.async_copy(tab.at[idxs_v.at[b]], rows[b], sems[b])
        pltpu.async_copy(dstq.at[start + b], idxd[b], isems[b])

    def _outer(ii, carry):
        for b in range(3):
            i = ii * 3 + b

            @pl.when(i < cnt)
            def _():
                pltpu.make_async_copy(tab.at[idxs_v.at[i]], rows[b],
                                      sems[b]).wait()
                pltpu.make_async_copy(dstq.at[start + i], idxd[b],
                                      isems[b]).wait()
                pltpu.sync_copy(rows[b], acc.at[idxd[b]], add=True)

                @pl.when(i + 3 < cnt)
                def _():
                    pltpu.async_copy(tab.at[idxs_v.at[i + 3]], rows[b],
                                     sems[b])
                    pltpu.async_copy(dstq.at[start + i + 3], idxd[b],
                                     isems[b])

        return carry

    lax.fori_loop(0, _OUTER, _outer, None)
    plsc.subcore_barrier()
    pltpu.sync_copy(acc.at[pl.ds(sid * _RPT, _RPT)],
                    outp.at[cid, pl.ds(sid * _RPT, _RPT)])

    @pl.when(sid == 0)
    def _():
        pltpu.sync_copy(acc.at[pl.ds(_RPT * _NS, _TAIL)],
                        outp.at[cid, pl.ds(_RPT * _NS, _TAIL)])


@functools.lru_cache(maxsize=None)
def _agg_call():
    mesh = plsc.VectorSubcoreMesh(core_axis_name="c", subcore_axis_name="s",
                                  num_cores=_NC, num_subcores=_NS)
    return pl.kernel(
        _agg_body,
        out_type=jax.ShapeDtypeStruct((_NC, N, 64), jnp.float32),
        mesh=mesh,
        compiler_params=pltpu.CompilerParams(use_tc_tiling_on_sc=False),
        scratch_types=[
            pltpu.VMEM((_ITERS, _CH), jnp.int32),
            pltpu.VMEM((_CH,), jnp.int32),
            pltpu.VMEM((_CH,), jnp.int32),
            pltpu.VMEM((_CH,), jnp.int32),
            pltpu.VMEM((_CH, 64), jnp.float32),
            pltpu.VMEM((_CH, 64), jnp.float32),
            pltpu.VMEM((_CH, 64), jnp.float32),
            pltpu.VMEM((_RPT, 64), jnp.float32),
            pltpu.VMEM_SHARED((N, 64), jnp.float32),
            pltpu.SemaphoreType.DMA,
            pltpu.SemaphoreType.DMA,
            pltpu.SemaphoreType.DMA,
            pltpu.SemaphoreType.DMA,
            pltpu.SemaphoreType.DMA,
            pltpu.SemaphoreType.DMA,
        ],
    )


def _leaky(v):
    return jnp.where(v >= 0.0, v, 0.01 * v)


_CB = 2048
_NPAD = 10240


def _lstm_kernel(nt0_ref, nt1_ref, w0_ref, w1_ref, whh_ref, bg_ref, hT_ref):
    n0 = nt0_ref[...]
    n1 = nt1_ref[...]
    w0 = w0_ref[...]
    w1 = w1_ref[...]
    whh = whh_ref[...]
    bg = bg_ref[...]
    h = jnp.zeros((HID, _CB), jnp.float32)
    c = jnp.zeros((HID, _CB), jnp.float32)
    for t in range(LSEQ):
        xt0 = lax.slice(n0, (t, 0), (t + 1, _CB))
        xt1 = lax.slice(n1, (t, 0), (t + 1, _CB))
        g = (w0 * xt0 + w1 * xt1 + bg
             + jnp.dot(whh, h, preferred_element_type=jnp.float32))
        ig = jax.nn.sigmoid(g[0:16])
        fg = jax.nn.sigmoid(g[16:32])
        gg = jnp.tanh(g[32:48])
        og = jax.nn.sigmoid(g[48:64])
        c = fg * c + ig * gg
        h = og * jnp.tanh(c)
    hT_ref[...] = h


def _stageB2_kernel(x_ref, rn_ref, degp_ref, w1x_ref, w1h_ref,
                    xs1_ref, self1_ref, dinv_ref):
    dp = degp_ref[...]
    deg = dp[0][:, 0:1] + dp[1][:, 0:1] + 1.0
    dinv = lax.rsqrt(deg)
    dinv_ref[...] = dinv
    xw = (jnp.dot(x_ref[...], w1x_ref[...], preferred_element_type=jnp.float32)
          + jnp.dot(rn_ref[...], w1h_ref[...],
                    preferred_element_type=jnp.float32))
    xs1_ref[...] = xw * dinv
    self1_ref[...] = xw * (dinv * dinv)


def _stageC_kernel(aggp_ref, self1_ref, dinv_ref, b1_ref, wg2_ref,
                   xs2_ref, self2_ref):
    ap = aggp_ref[...]
    dinv = dinv_ref[...]
    pre = dinv * (ap[0] + ap[1]) + self1_ref[...] + b1_ref[...]
    out1 = _leaky(pre)
    xw2 = jnp.dot(out1, wg2_ref[...], preferred_element_type=jnp.float32)
    xs2_ref[...] = xw2 * dinv
    self2_ref[...] = xw2 * (dinv * dinv)


def _stageD_kernel(aggp_ref, self2_ref, dinv_ref, b2_ref, x_ref, rn_ref,
                   wcx_ref, wch_ref, wcg_ref, b1_ref, w2T_ref, b2c_ref,
                   w3T_ref, b3_ref, out_ref, xgcn_ref):
    ap = aggp_ref[...]
    dinv = dinv_ref[...]
    pre = dinv * (ap[0] + ap[1]) + self2_ref[...] + b2_ref[...]
    xg = _leaky(pre)
    xgcn_ref[...] = xg
    h1 = _leaky(jnp.dot(x_ref[...], wcx_ref[...],
                        preferred_element_type=jnp.float32)
                + jnp.dot(rn_ref[...], wch_ref[...],
                          preferred_element_type=jnp.float32)
                + jnp.dot(xg, wcg_ref[...],
                          preferred_element_type=jnp.float32)
                + b1_ref[...])
    h2 = _leaky(jnp.dot(h1, w2T_ref[...], preferred_element_type=jnp.float32)
                + b2c_ref[...])
    out_ref[...] = (jnp.dot(h2, w3T_ref[...], preferred_element_type=jnp.float32)
                    + b3_ref[...])


def _row_spec(cols):
    return pl.BlockSpec((_RB, cols), lambda i: (i, 0))


def _full_spec(shape):
    nd = len(shape)
    return pl.BlockSpec(shape, lambda i, _n=nd: (0,) * _n)


def _part_spec(cols):
    return pl.BlockSpec((_NC, _RB, cols), lambda i: (0, i, 0))


def _lstm(nt0T, nt1T, w0, w1, whh, bgc):
    return pl.pallas_call(
        _lstm_kernel,
        grid=(_NPAD // _CB,),
        in_specs=[
            pl.BlockSpec((LSEQ, _CB), lambda i: (0, i)),
            pl.BlockSpec((LSEQ, _CB), lambda i: (0, i)),
            _full_spec((4 * HID, 1)), _full_spec((4 * HID, 1)),
            _full_spec((4 * HID, HID)), _full_spec((4 * HID, 1)),
        ],
        out_specs=pl.BlockSpec((HID, _CB), lambda i: (0, i)),
        out_shape=jax.ShapeDtypeStruct((HID, _NPAD), jnp.float32),
    )(nt0T, nt1T, w0, w1, whh, bgc)


def _stageB2(x, rn, degp, w1x, w1h):
    return pl.pallas_call(
        _stageB2_kernel,
        grid=(_GRID,),
        in_specs=[
            _row_spec(DF), _row_spec(HID), _part_spec(16),
            _full_spec((DF, 64)), _full_spec((HID, 64)),
        ],
        out_specs=[_row_spec(64), _row_spec(64), _row_spec(1)],
        out_shape=[
            jax.ShapeDtypeStruct((N, 64), jnp.float32),
            jax.ShapeDtypeStruct((N, 64), jnp.float32),
            jax.ShapeDtypeStruct((N, 1), jnp.float32),
        ],
    )(x, rn, degp, w1x, w1h)


def _stageC(aggp, self1, dinv, b1, wg2):
    return pl.pallas_call(
        _stageC_kernel,
        grid=(_GRID,),
        in_specs=[_part_spec(64), _row_spec(64), _row_spec(1),
                  _full_spec((1, 64)), _full_spec((64, 64))],
        out_specs=[_row_spec(64), _row_spec(64)],
        out_shape=[
            jax.ShapeDtypeStruct((N, 64), jnp.float32),
            jax.ShapeDtypeStruct((N, 64), jnp.float32),
        ],
    )(aggp, self1, dinv, b1, wg2)


def _stageD(aggp, self2, dinv, b2, x, rn, wcx, wch, wcg, b1, w2T, b2c,
            w3T, b3):
    return pl.pallas_call(
        _stageD_kernel,
        grid=(_GRID,),
        in_specs=[_part_spec(64), _row_spec(64), _row_spec(1),
                  _full_spec((1, 64)), _row_spec(DF), _row_spec(HID),
                  _full_spec((DF, 32)), _full_spec((HID, 32)),
                  _full_spec((64, 32)), _full_spec((1, 32)),
                  _full_spec((32, 16)), _full_spec((1, 16)),
                  _full_spec((16, 4)), _full_spec((1, 4))],
        out_specs=[_row_spec(4), _row_spec(64)],
        out_shape=[
            jax.ShapeDtypeStruct((N, 4), jnp.float32),
            jax.ShapeDtypeStruct((N, 64), jnp.float32),
        ],
    )(aggp, self2, dinv, b2, x, rn, wcx, wch, wcg, b1, w2T, b2c, w3T, b3)


def kernel(x, edge_data, node_trans, adj_mat, W_ih, W_hh, b_ih, b_hh,
           W_dnn, b_dnn, W_g1, att1, bias1, W_g2, att2, bias2,
           W_gcn1, b_gcn1, W_gcn2, b_gcn2, W_cls1, b_cls1, W_cls2, b_cls2,
           W_cls3, b_cls3, edge_index):
    srcq = jnp.pad(edge_index[0].reshape(_NCHUNK, _CH),
                   ((0, _QPAD - _NCHUNK), (0, 0)))
    dstq = jnp.pad(edge_index[1].reshape(_NCHUNK, _CH),
                   ((0, _QPAD - _NCHUNK), (0, 0)))
    nt0T = jnp.pad(node_trans[:, :, 0].T, ((0, 0), (0, _NPAD - N)))
    nt1T = jnp.pad(node_trans[:, :, 1].T, ((0, 0), (0, _NPAD - N)))
    w0 = W_ih[:, 0:1]
    w1 = W_ih[:, 1:2]
    bgc = (b_ih + b_hh).reshape(4 * HID, 1)
    wcls1T = W_cls1.T

    degp = _deg_call()(dstq)
    rn = _lstm(nt0T, nt1T, w0, w1, W_hh, bgc)[:, :N].T
    xs1, self1, dinv = _stageB2(x, rn, degp, W_gcn1[:DF], W_gcn1[DF:])
    agg1 = _agg_call()(srcq, dstq, xs1)
    xs2, self2 = _stageC(agg1, self1, dinv, b_gcn1.reshape(1, 64), W_gcn2)
    agg2 = _agg_call()(srcq, dstq, xs2)
    out, xgcn = _stageD(agg2, self2, dinv, b_gcn2.reshape(1, 64), x, rn,
                        wcls1T[:DF], wcls1T[DF:DF + HID],
                        wcls1T[DF + HID:], b_cls1.reshape(1, 32),
                        W_cls2.T, b_cls2.reshape(1, 16),
                        W_cls3.T, b_cls3.reshape(1, 4))
    return (out, xgcn, adj_mat)
